# Initial kernel scaffold; baseline (speedup 1.0000x reference)
import jax
import jax.numpy as jnp
from jax.experimental import pallas as pl


def kernel(nodes, edges, poses):
    raise NotImplementedError("write your pallas kernel here")


# speedup vs baseline: 5.9062x; 5.9062x over previous
import jax
import jax.numpy as jnp
from jax import lax
from jax.experimental import pallas as pl
from jax.experimental.pallas import tpu as pltpu
from jax.experimental.pallas import tpu_sc as plsc

_N_NODES = 100000
_N_NODES_PAD = 100352
_N_EDGES = 3200000
_NC = 2
_NS = 16
_NW = _NC * _NS
_E = 1024
_NCHUNK = _N_EDGES // _E
_G = _E // 16

_HALF_PI = 1.5707963267948966
_C_ATAN = (0.9999998977903125, -0.33331959846433457, 0.19969236562476794,
           -0.1401658988294469, 0.09906106970472799, -0.059367209648048674,
           0.02416624674527195, -0.004668784473913435)


def _rsqrt(x):
    i = plsc.bitcast(x, jnp.int32)
    i = jnp.int32(0x5F3759DF) - lax.shift_right_logical(i, 1)
    y = plsc.bitcast(i, jnp.float32)
    for _ in range(3):
        y = y * (1.5 - 0.5 * x * y * y)
    return y


def _qmul(q, r):
    qx, qy, qz, qw = q
    rx, ry, rz, rw = r
    return [qw * rx + qx * rw + qy * rz - qz * ry,
            qw * ry - qx * rz + qy * rw + qz * rx,
            qw * rz + qx * ry - qy * rx + qz * rw,
            qw * rw - qx * rx - qy * ry - qz * rz]


def _qconj(q):
    return [-q[0], -q[1], -q[2], q[3]]


def _cross(a, b):
    return [a[1] * b[2] - a[2] * b[1],
            a[2] * b[0] - a[0] * b[2],
            a[0] * b[1] - a[1] * b[0]]


def _qrot(q, v):
    qv = q[:3]
    w = q[3]
    t = [2.0 * c for c in _cross(qv, v)]
    ct = _cross(qv, t)
    return [v[i] + w * t[i] + ct[i] for i in range(3)]


def _edge_math(t1, q1, t2, q2, tp, qp):
    qA = _qmul(q2, _qconj(q1))
    rA = _qrot(qA, t1)
    tA = [t2[i] - rA[i] for i in range(3)]
    qip = _qconj(qp)
    qe = _qmul(qip, qA)
    d = [tA[i] - tp[i] for i in range(3)]
    te = _qrot(qip, d)

    w0 = qe[3]
    sflip = jnp.where(w0 < 0.0, jnp.float32(-1.0), jnp.float32(1.0))
    qv = [qe[i] * sflip for i in range(3)]
    w = jnp.abs(w0)
    n2q = qv[0] * qv[0] + qv[1] * qv[1] + qv[2] * qv[2]
    n = n2q * _rsqrt(jnp.maximum(n2q, jnp.float32(1e-30)))
    num = jnp.minimum(n, w)
    den = jnp.maximum(n, w)
    t = num / den
    u = t * t
    a = jnp.float32(_C_ATAN[-1])
    for ck in _C_ATAN[-2::-1]:
        a = a * u + jnp.float32(ck)
    a = a * t
    half = jnp.where(n > w, jnp.float32(_HALF_PI) - a, a)
    theta = 2.0 * half
    eps = jnp.float32(1e-7)
    big_n = n > eps
    w_safe = jnp.where(w > eps, w, jnp.float32(1.0))
    scale = jnp.where(big_n, theta, jnp.float32(2.0)) / jnp.where(big_n, n, w_safe)
    phi = [qv[i] * scale for i in range(3)]
    small = theta < 1e-4
    s = 2.0 * n * w
    c_ = w * w - n2q
    th2 = theta * theta
    denom = 2.0 * theta * s
    denom = jnp.where(jnp.abs(denom) > 1e-12, denom, jnp.float32(1e-12))
    coef = (denom - (1.0 + c_) * th2) / (th2 * denom)
    coef = jnp.where(small, jnp.float32(1.0 / 12.0), coef)
    pxt = _cross(phi, te)
    cpp = _cross(phi, pxt)
    rho = [te[i] - 0.5 * pxt[i] + coef * cpp[i] for i in range(3)]
    return rho + phi


def _sc_body(ncm_hbm, e0_hbm, e1_hbm, poses_hbm, out_hbm,
             tab, idx0_v, idx1_v, nd1, nd2, p_v, o_v, sem):
    c = lax.axis_index("c")
    s = lax.axis_index("s")
    wid = s * _NC + c

    per_tile = _N_NODES_PAD // _NS
    r0 = s * per_tile
    for comp in range(7):
        pltpu.sync_copy(ncm_hbm[comp].at[pl.ds(r0, per_tile)],
                        tab[comp].at[pl.ds(r0, per_tile)])
    plsc.subcore_barrier()

    lane = lax.iota(jnp.int32, 16)
    lane7 = lane * 7
    lane6 = lane * 6
    n_chunks = (_NCHUNK - wid + _NW - 1) // _NW

    def chunk_body(k, carry):
        goff = (wid + k * _NW) * _E
        pltpu.sync_copy(e0_hbm.at[pl.ds(goff, _E)], idx0_v)
        pltpu.sync_copy(e1_hbm.at[pl.ds(goff, _E)], idx1_v)
        cps = []
        for comp in range(7):
            cps.append(pltpu.async_copy(tab[comp].at[idx0_v], nd1[comp], sem))
            cps.append(pltpu.async_copy(tab[comp].at[idx1_v], nd2[comp], sem))
        pltpu.sync_copy(poses_hbm.at[pl.ds(goff * 7, _E * 7)], p_v)
        for cp in cps:
            cp.wait()

        def group_body(j, carry2):
            e16 = pl.ds(j * 16, 16)
            pbase = j * 112 + lane7
            t1 = [nd1[i][e16] for i in range(3)]
            q1 = [nd1[3 + i][e16] for i in range(4)]
            t2 = [nd2[i][e16] for i in range(3)]
            q2 = [nd2[3 + i][e16] for i in range(4)]
            tp = [plsc.load_gather(p_v, [pbase + i]) for i in range(3)]
            qp = [plsc.load_gather(p_v, [pbase + 3 + i]) for i in range(4)]
            res = _edge_math(t1, q1, t2, q2, tp, qp)
            obase = j * 96 + lane6
            for i in range(6):
                plsc.store_scatter(o_v, [obase + i], res[i])
            return carry2

        lax.fori_loop(0, _G, group_body, 0)
        pltpu.sync_copy(o_v, out_hbm.at[pl.ds(goff * 6, _E * 6)])
        return carry

    lax.fori_loop(0, n_chunks, chunk_body, 0)


@jax.jit
def _pose_graph_sc(nodes_cm, e0, e1, poses_f):
    run = pl.kernel(
        _sc_body,
        out_type=jax.ShapeDtypeStruct((_N_EDGES * 6,), jnp.float32),
        mesh=plsc.VectorSubcoreMesh(core_axis_name="c", subcore_axis_name="s"),
        compiler_params=pltpu.CompilerParams(needs_layout_passes=False),
        scratch_types=[
            [pltpu.VMEM_SHARED((_N_NODES_PAD,), jnp.float32) for _ in range(7)],
            pltpu.VMEM((_E,), jnp.int32),
            pltpu.VMEM((_E,), jnp.int32),
            [pltpu.VMEM((_E,), jnp.float32) for _ in range(7)],
            [pltpu.VMEM((_E,), jnp.float32) for _ in range(7)],
            pltpu.VMEM((_E * 7,), jnp.float32),
            pltpu.VMEM((_E * 6,), jnp.float32),
            pltpu.SemaphoreType.DMA,
        ],
    )
    return run(nodes_cm, e0, e1, poses_f)


def kernel(nodes, edges, poses):
    padded = jnp.pad(nodes.T, ((0, 0), (0, _N_NODES_PAD - _N_NODES)))
    nodes_cm = tuple(padded[i] for i in range(7))
    e0 = edges[:, 0]
    e1 = edges[:, 1]
    poses_f = poses.reshape(-1)
    out = _pose_graph_sc(nodes_cm, e0, e1, poses_f)
    return out.reshape(_N_EDGES, 6)
